# baseline (speedup 1.0000x reference)
import functools

import jax
import jax.numpy as jnp
from jax import lax
from jax.experimental import pallas as pl
from jax.experimental.pallas import tpu as pltpu
from jax.experimental.pallas import tpu_sc as plsc

NC = 2
NS = 16
NW = NC * NS
CHUNK = 128
R = 2
Q = 4


@functools.partial(jax.jit, static_argnums=(2, 3))
def _sc_gather(table, idx, n_chunks, emb_dim):
    n_rows = NW * n_chunks * CHUNK

    mesh = plsc.VectorSubcoreMesh(
        core_axis_name="c", subcore_axis_name="s",
        num_cores=NC, num_subcores=NS,
    )

    @functools.partial(
        pl.kernel,
        out_type=jax.ShapeDtypeStruct((n_rows, emb_dim), jnp.float32),
        mesh=mesh,
        scratch_types=(
            [
                pltpu.VMEM((n_chunks, CHUNK), jnp.int32),
                pltpu.VMEM_SHARED((NS, Q, CHUNK, emb_dim), jnp.float32),
            ]
            + [pltpu.VMEM((CHUNK, emb_dim), jnp.float32) for _ in range(R)]
            + [pltpu.SemaphoreType.DMA for _ in range(2 * R + Q)]
        ),
    )
    def k(table_hbm, idx_hbm, out_hbm, idx_v, spst, *rest):
        rows = rest[:R]
        gsem = rest[R:2 * R]
        csem = rest[2 * R:3 * R]
        wsem = rest[3 * R:3 * R + Q]
        cid = lax.axis_index("c")
        sid = lax.axis_index("s")
        wid = sid * NC + cid
        pltpu.sync_copy(idx_hbm.at[wid], idx_v)
        base = wid * (n_chunks * CHUNK)

        def gdesc(j, b):
            return pltpu.make_async_copy(
                table_hbm.at[idx_v.at[j]], rows[b], gsem[b])

        def cdesc(b, q):
            return pltpu.make_async_copy(rows[b], spst.at[sid, q], csem[b])

        def wdesc(j, q):
            return pltpu.make_async_copy(
                spst.at[sid, q],
                out_hbm.at[pl.ds(base + j * CHUNK, CHUNK)],
                wsem[q],
            )

        gdesc(0, 0).start()
        gdesc(1, 1).start()

        @pl.loop(0, n_chunks, step=Q)
        def _(g):
            for q in range(Q):
                j = g + q
                b = q % R

                @pl.when(j >= Q)
                def _():
                    wdesc(j - Q, q).wait()

                gdesc(j, b).wait()
                cdesc(b, q).start()
                cdesc(b, q).wait()
                wdesc(j, q).start()

                @pl.when(j + 2 < n_chunks)
                def _():
                    gdesc(j + 2, b).start()

        for j in range(n_chunks - Q, n_chunks):
            wdesc(j, j % Q).wait()

    return k(table, idx)


def kernel(indices, embedding_weight):
    b, s = indices.shape
    v, d = embedding_weight.shape
    n = b * s
    n_chunks = n // (NW * CHUNK)
    idx = indices.astype(jnp.int32).reshape(NW, n_chunks, CHUNK)
    out = _sc_gather(embedding_weight, idx, n_chunks, d)
    return out.reshape(b, s, d)


# speedup vs baseline: 1.0211x; 1.0211x over previous
import functools

import jax
import jax.numpy as jnp
from jax import lax
from jax.experimental import pallas as pl
from jax.experimental.pallas import tpu as pltpu
from jax.experimental.pallas import tpu_sc as plsc

NC = 2
NS = 16
NW = NC * NS
CHUNK = 128
G4 = 4
NSP = 3


@functools.partial(jax.jit, static_argnums=(2, 3))
def _sc_gather(table, idx, n_chunks, emb_dim):
    n_rows = NW * n_chunks * CHUNK
    n_groups = n_chunks // G4

    mesh = plsc.VectorSubcoreMesh(
        core_axis_name="c", subcore_axis_name="s",
        num_cores=NC, num_subcores=NS,
    )

    @functools.partial(
        pl.kernel,
        out_type=jax.ShapeDtypeStruct((n_rows, emb_dim), jnp.float32),
        mesh=mesh,
        scratch_types=(
            [
                pltpu.VMEM((2, G4, CHUNK), jnp.int32),
                pltpu.VMEM_SHARED((NS, NSP, CHUNK, emb_dim), jnp.float32),
            ]
            + [pltpu.VMEM((CHUNK, emb_dim), jnp.float32) for _ in range(G4)]
            + [pltpu.SemaphoreType.DMA for _ in range(2)]
            + [pltpu.SemaphoreType.DMA for _ in range(G4)]
            + [pltpu.SemaphoreType.DMA for _ in range(NSP)]
            + [pltpu.SemaphoreType.DMA for _ in range(NSP)]
            + [pltpu.SemaphoreType.DMA]
        ),
    )
    def k(table_hbm, idx_hbm, out_hbm, ibuf, spst, *rest):
        rows = rest[:G4]
        isem = rest[G4:G4 + 2]
        gsem = rest[G4 + 2:2 * G4 + 2]
        csem = rest[2 * G4 + 2:2 * G4 + 2 + NSP]
        wsem = rest[2 * G4 + 2 + NSP:2 * G4 + 2 + 2 * NSP]
        dsem = rest[2 * G4 + 2 + 2 * NSP]
        cid = lax.axis_index("c")
        sid = lax.axis_index("s")
        wid = sid * NC + cid
        base = wid * (n_chunks * CHUNK)

        def idesc(grp, sl):
            return pltpu.make_async_copy(
                idx_hbm.at[wid, pl.ds(grp * G4, G4)], ibuf.at[sl], isem[sl])

        def gdesc(j, grp_sl, q):
            return pltpu.make_async_copy(
                table_hbm.at[ibuf.at[grp_sl, q]], rows[q], gsem[q])

        def cdesc(q):
            return pltpu.make_async_copy(rows[q], spst.at[sid, q], csem[q])

        def wdesc(j, q):
            return pltpu.make_async_copy(
                spst.at[sid, q],
                out_hbm.at[pl.ds(base + j * CHUNK, CHUNK)],
                wsem[q],
            )

        def ddesc(j):
            return pltpu.make_async_copy(
                rows[3],
                out_hbm.at[pl.ds(base + j * CHUNK, CHUNK)],
                dsem,
            )

        idesc(0, 0).start()
        idesc(1, 1).start()
        idesc(0, 0).wait()
        for q in range(G4):
            gdesc(q, 0, q).start()

        @pl.loop(0, n_groups, step=2)
        def _(go):
            for par in range(2):
                g = go + par
                sl = par
                nsl = 1 - par
                j0 = g * G4

                @pl.when(g + 1 < n_groups)
                def _():
                    idesc(g + 1, nsl).wait()

                for q in range(NSP):
                    j = j0 + q

                    @pl.when(j >= G4)
                    def _():
                        wdesc(j - G4, q).wait()

                    gdesc(j, sl, q).wait()
                    cdesc(q).start()
                    cdesc(q).wait()
                    wdesc(j, q).start()

                    @pl.when(j + G4 < n_chunks)
                    def _():
                        gdesc(j + G4, nsl, q).start()

                j = j0 + 3
                gdesc(j, sl, 3).wait()
                ddesc(j).start()

                @pl.when(j + G4 < n_chunks)
                def _():
                    ddesc(j).wait()
                    gdesc(j + G4, nsl, 3).start()

                @pl.when(g + 2 < n_groups)
                def _():
                    idesc(g + 2, sl).start()

        for j in range(n_chunks - G4, n_chunks - 1):
            wdesc(j, j % G4).wait()
        ddesc(n_chunks - 1).wait()

    return k(table, idx)


def kernel(indices, embedding_weight):
    b, s = indices.shape
    v, d = embedding_weight.shape
    n = b * s
    n_chunks = n // (NW * CHUNK)
    idx = indices.astype(jnp.int32).reshape(NW, n_chunks, CHUNK)
    out = _sc_gather(embedding_weight, idx, n_chunks, d)
    return out.reshape(b, s, d)
